# baseline (speedup 1.0000x reference)
import functools

import jax
import jax.numpy as jnp
from jax import lax
from jax.experimental import pallas as pl
from jax.experimental.pallas import tpu as pltpu
from jax.experimental.pallas import tpu_sc as plsc

VOCAB = 1000000
EMBED_DIM = 64

NUM_CORES = 2
NUM_SUBCORES = 16
NUM_WORKERS = NUM_CORES * NUM_SUBCORES

IDX_MINOR = 128
K = 8
GROUP = K * IDX_MINOR


def _make_kernel(n_lookups):
    assert n_lookups % (NUM_WORKERS * GROUP) == 0
    rows_per_worker = n_lookups // (NUM_WORKERS * IDX_MINOR)
    groups = rows_per_worker // K

    mesh = plsc.VectorSubcoreMesh(
        core_axis_name="c", subcore_axis_name="s",
        num_cores=NUM_CORES, num_subcores=NUM_SUBCORES)

    @functools.partial(
        pl.kernel,
        out_type=jax.ShapeDtypeStruct((n_lookups, EMBED_DIM), jnp.float32),
        mesh=mesh,
        scratch_types=[
            pltpu.VMEM((K, IDX_MINOR), jnp.int32),
            pltpu.VMEM((GROUP, EMBED_DIM), jnp.float32),
            pltpu.SemaphoreType.DMA,
        ],
        compiler_params=pltpu.CompilerParams(use_tc_tiling_on_sc=False),
    )
    def body(x_hbm, table_hbm, out_hbm, idx_v, rows_v, sem):
        wid = lax.axis_index("s") * NUM_CORES + lax.axis_index("c")
        row_base = wid * rows_per_worker

        def group(g, carry):
            row0 = row_base + g * K
            pltpu.sync_copy(x_hbm.at[pl.ds(row0, K)], idx_v)
            copies = []
            for j in range(K):
                copies.append(pltpu.async_copy(
                    table_hbm.at[idx_v.at[j]],
                    rows_v.at[pl.ds(j * IDX_MINOR, IDX_MINOR)],
                    sem))
            for c in copies:
                c.wait()
            pltpu.sync_copy(rows_v,
                            out_hbm.at[pl.ds(row0 * IDX_MINOR, GROUP)])
            return carry

        lax.fori_loop(0, groups, group, 0)

    return body


def kernel(x, table):
    b, s = x.shape
    n = b * s
    x_flat = x.reshape(n // IDX_MINOR, IDX_MINOR)
    out = _make_kernel(n)(x_flat, table)
    return out.reshape(b, s, EMBED_DIM)
